# baseline (speedup 1.0000x reference)
import jax
import jax.numpy as jnp
from jax import lax
from jax.experimental import pallas as pl
from jax.experimental.pallas import tpu as pltpu
from jax.experimental.pallas import tpu_sc as plsc

N = 10000
E = 320000
D = 128
HID = 128
H = 8
DH = 128

NC = 2
NS = 16
L = 16
NW = NC * NS
NPAD = 10240
ROWS_PER_SUB = NPAD // NS
CHUNK = 64
TCHP = -(-E // (CHUNK * NW)) * NW
EPAD = TCHP * CHUNK
TCHX = TCHP + 2 * NW
TRIPS_W = TCHP // NW
HPC = H // NC

CH2 = 128
TCP2 = -(-E // (CH2 * NS)) * NS
TCP2 = -(-TCP2 // (2 * NS)) * (2 * NS)
EPAD2 = TCP2 * CH2
TCX2 = TCP2 + 2 * NS
TRIPS_S = TCP2 // NS
EEROWS = 2 * TCX2

_MESH = plsc.VectorSubcoreMesh(
    core_axis_name="c", subcore_axis_name="s", num_cores=NC, num_subcores=NS)

_TAKE_DNUMS = lax.GatherDimensionNumbers(
    offset_dims=(), collapsed_slice_dims=(0,), start_index_map=(0,))


def _take16(vec, idx):
    return lax.gather(vec, idx[:, None], _TAKE_DNUMS, (1,),
                      mode=lax.GatherScatterMode.PROMISE_IN_BOUNDS)


def _zero_rows(buf, width):
    z = jnp.zeros((L,), jnp.float32)

    @pl.loop(0, buf.shape[0])
    def _(i):
        for t in range(width // L):
            buf[i, pl.ds(t * L, L)] = z


def _zero_slice(zeros_v, shared, base_row):
    @pl.loop(0, ROWS_PER_SUB, step=CHUNK)
    def _(r):
        pltpu.sync_copy(zeros_v, shared.at[pl.ds(base_row + r, CHUNK)])


def _copy_out(shared, out_hbm, base_row, out_base):
    @pl.loop(0, ROWS_PER_SUB, step=CHUNK)
    def _(r):
        pltpu.sync_copy(shared.at[pl.ds(base_row + r, CHUNK)],
                        out_hbm.at[pl.ds(out_base + r, CHUNK)])


def _dst_half(ids_v, dst_v):
    for q in range(4):
        dst_v[pl.ds(q * L, L)] = ids_v[pl.ds(CHUNK + q * L, L)]


def _src_slice(ids_v):
    return ids_v.at[pl.ds(0, CHUNK)]


def _deg_body(ids_hbm, deg_hbm, acc_sh,
              ids_v0, ids_v1, dst_v0, dst_v1, ones_v, semL0, semL1):
    c = lax.axis_index("c")
    s = lax.axis_index("s")
    base_row = s * ROWS_PER_SUB
    _zero_rows(ones_v, D)
    _zero_slice(ones_v, acc_sh, base_row)
    one0 = jnp.where(lax.iota(jnp.int32, L) == 0,
                     jnp.float32(1.0), jnp.float32(0.0))

    @pl.loop(0, CHUNK)
    def _(i):
        ones_v[i, pl.ds(0, L)] = one0

    plsc.subcore_barrier()
    w = c * NS + s
    ids_b = (ids_v0, ids_v1)
    dst_b = (dst_v0, dst_v1)
    semL = (semL0, semL1)

    pltpu.async_copy(ids_hbm.at[w], ids_v0, semL0)
    pltpu.async_copy(ids_hbm.at[w + NW], ids_v1, semL1)

    @pl.loop(0, TRIPS_W // 2)
    def _(i2):
        for b in range(2):
            t = i2 * 2 + b
            cid = w + t * NW
            pltpu.make_async_copy(ids_hbm.at[cid], ids_b[b], semL[b]).wait()
            _dst_half(ids_b[b], dst_b[b])
            pltpu.async_copy(ids_hbm.at[cid + 2 * NW], ids_b[b], semL[b])
            pltpu.sync_copy(ones_v, acc_sh.at[dst_b[b]], add=True)

    t = TRIPS_W - 1
    cid = w + t * NW
    pltpu.make_async_copy(ids_hbm.at[cid], ids_v0, semL0).wait()
    pltpu.make_async_copy(ids_hbm.at[cid + NW], ids_v1, semL1).wait()
    _dst_half(ids_v0, dst_v0)
    pltpu.sync_copy(ones_v, acc_sh.at[dst_v0], add=True)

    plsc.subcore_barrier()
    _copy_out(acc_sh, deg_hbm, base_row, c * NPAD + base_row)


def _sc_deg(ids2):
    fn = pl.kernel(
        _deg_body,
        out_type=jax.ShapeDtypeStruct((NC * NPAD, D), jnp.float32),
        mesh=_MESH,
        scratch_types=[
            pltpu.VMEM_SHARED((NPAD, D), jnp.float32),
            pltpu.VMEM((2 * CHUNK,), jnp.int32),
            pltpu.VMEM((2 * CHUNK,), jnp.int32),
            pltpu.VMEM((CHUNK,), jnp.int32),
            pltpu.VMEM((CHUNK,), jnp.int32),
            pltpu.VMEM((CHUNK, D), jnp.float32),
            pltpu.SemaphoreType.DMA,
            pltpu.SemaphoreType.DMA,
        ],
    )
    return fn(ids2)


def _gin_agg_body(ids_hbm, h_hbm, ssum_hbm, acc_sh,
                  ids_v0, ids_v1, dst_v0, dst_v1, rows_v0, rows_v1,
                  zeros_v, semL0, semL1, semG0, semG1):
    c = lax.axis_index("c")
    s = lax.axis_index("s")
    base_row = s * ROWS_PER_SUB
    _zero_rows(zeros_v, D)
    _zero_slice(zeros_v, acc_sh, base_row)
    plsc.subcore_barrier()

    w = c * NS + s
    ids_b = (ids_v0, ids_v1)
    dst_b = (dst_v0, dst_v1)
    rows_b = (rows_v0, rows_v1)
    semL = (semL0, semL1)
    semG = (semG0, semG1)

    pltpu.async_copy(ids_hbm.at[w], ids_v0, semL0).wait()
    pltpu.async_copy(h_hbm.at[_src_slice(ids_v0)], rows_v0, semG0)
    pltpu.async_copy(ids_hbm.at[w + NW], ids_v1, semL1)

    def trip(t_expr, b):
        nb = 1 - b
        cid = w + t_expr * NW
        pltpu.make_async_copy(ids_hbm.at[cid + NW], ids_b[nb],
                              semL[nb]).wait()
        pltpu.async_copy(h_hbm.at[_src_slice(ids_b[nb])], rows_b[nb],
                         semG[nb])
        pltpu.make_async_copy(h_hbm.at[_src_slice(ids_b[b])], rows_b[b],
                              semG[b]).wait()
        _dst_half(ids_b[b], dst_b[b])
        pltpu.sync_copy(rows_b[b], acc_sh.at[dst_b[b]], add=True)
        pltpu.async_copy(ids_hbm.at[cid + 2 * NW], ids_b[b], semL[b])

    @pl.loop(0, TRIPS_W // 2)
    def _(i2):
        trip(i2 * 2, 0)
        trip(i2 * 2 + 1, 1)

    t = TRIPS_W - 1
    cid = w + t * NW
    pltpu.make_async_copy(ids_hbm.at[cid + NW], ids_v1, semL1).wait()
    pltpu.make_async_copy(h_hbm.at[_src_slice(ids_v0)], rows_v0,
                          semG0).wait()
    _dst_half(ids_v0, dst_v0)
    pltpu.sync_copy(rows_v0, acc_sh.at[dst_v0], add=True)

    plsc.subcore_barrier()
    _copy_out(acc_sh, ssum_hbm, base_row, c * NPAD + base_row)


def _sc_gin_agg(ids2, h):
    fn = pl.kernel(
        _gin_agg_body,
        out_type=jax.ShapeDtypeStruct((NC * NPAD, D), jnp.float32),
        mesh=_MESH,
        scratch_types=[
            pltpu.VMEM_SHARED((NPAD, D), jnp.float32),
            pltpu.VMEM((2 * CHUNK,), jnp.int32),
            pltpu.VMEM((2 * CHUNK,), jnp.int32),
            pltpu.VMEM((CHUNK,), jnp.int32),
            pltpu.VMEM((CHUNK,), jnp.int32),
            pltpu.VMEM((CHUNK, D), jnp.float32),
            pltpu.VMEM((CHUNK, D), jnp.float32),
            pltpu.VMEM((CHUNK, D), jnp.float32),
            pltpu.SemaphoreType.DMA,
            pltpu.SemaphoreType.DMA,
            pltpu.SemaphoreType.DMA,
            pltpu.SemaphoreType.DMA,
        ],
    )
    return fn(ids2, h)


def _gat_edge_body(ids_hbm, elt_hbm, ert_hbm, ee_hbm, z_hbm, z_sh,
                   ids_v0, ids_v1, dst_v0, dst_v1, s_v0, s_v1, d_v0, d_v1,
                   ee_v, zrow_v,
                   semL0, semL1, semG0, semG1, semH0, semH1):
    c = lax.axis_index("c")
    s = lax.axis_index("s")
    base_row = s * ROWS_PER_SUB
    _zero_rows(zrow_v, D)
    _zero_slice(zrow_v, z_sh, base_row)
    plsc.subcore_barrier()
    rot = (lax.iota(jnp.int32, L) + 8) & 15
    w = c * NS + s

    ids_b = (ids_v0, ids_v1)
    dst_b = (dst_v0, dst_v1)
    s_b = (s_v0, s_v1)
    d_b = (d_v0, d_v1)
    semL = (semL0, semL1)
    semG = (semG0, semG1)
    semH = (semH0, semH1)

    def issue_gathers(b):
        pltpu.async_copy(elt_hbm.at[_src_slice(ids_b[b])], s_b[b], semG[b])
        _dst_half(ids_b[b], dst_b[b])
        pltpu.async_copy(ert_hbm.at[dst_b[b]], d_b[b], semH[b])

    pltpu.async_copy(ids_hbm.at[w], ids_v0, semL0).wait()
    issue_gathers(0)
    pltpu.async_copy(ids_hbm.at[w + NW], ids_v1, semL1)

    def trip(t_expr, b):
        nb = 1 - b
        cid = w + t_expr * NW
        pltpu.make_async_copy(ids_hbm.at[cid + NW], ids_b[nb],
                              semL[nb]).wait()
        issue_gathers(nb)
        pltpu.make_async_copy(elt_hbm.at[_src_slice(ids_b[b])], s_b[b],
                              semG[b]).wait()
        pltpu.make_async_copy(ert_hbm.at[dst_b[b]], d_b[b], semH[b]).wait()

        @pl.loop(0, CHUNK, step=8)
        def _(jo):
            g = jo >> 3
            for t in range(8):
                srow = s_b[b][jo + t, pl.ds(0, L)]
                drow = d_b[b][jo + t, pl.ds(0, L)]
                esum = srow + drow
                e = jnp.where(esum >= 0, esum, 0.2 * esum)
                shift = _take16(drow, rot)
                ee = jnp.exp(e - shift)
                zrow_v[jo + t, pl.ds(0, L)] = ee
                ee_v[g, pl.ds(t * L, L)] = ee

        pltpu.sync_copy(ee_v, ee_hbm.at[cid])
        pltpu.sync_copy(zrow_v, z_sh.at[dst_b[b]], add=True)
        pltpu.async_copy(ids_hbm.at[cid + 2 * NW], ids_b[b], semL[b])

    @pl.loop(0, TRIPS_W // 2)
    def _(i2):
        trip(i2 * 2, 0)
        trip(i2 * 2 + 1, 1)

    t = TRIPS_W - 1
    cid = w + t * NW
    pltpu.make_async_copy(ids_hbm.at[cid + NW], ids_v1, semL1).wait()
    pltpu.make_async_copy(elt_hbm.at[_src_slice(ids_v0)], s_v0,
                          semG0).wait()
    pltpu.make_async_copy(ert_hbm.at[dst_v0], d_v0, semH0).wait()

    @pl.loop(0, CHUNK, step=8)
    def _(jo):
        g = jo >> 3
        for tt in range(8):
            srow = s_v0[jo + tt, pl.ds(0, L)]
            drow = d_v0[jo + tt, pl.ds(0, L)]
            esum = srow + drow
            e = jnp.where(esum >= 0, esum, 0.2 * esum)
            shift = _take16(drow, rot)
            ee = jnp.exp(e - shift)
            zrow_v[jo + tt, pl.ds(0, L)] = ee
            ee_v[g, pl.ds(tt * L, L)] = ee

    pltpu.sync_copy(ee_v, ee_hbm.at[cid])
    pltpu.sync_copy(zrow_v, z_sh.at[dst_v0], add=True)

    plsc.subcore_barrier()
    _copy_out(z_sh, z_hbm, base_row, c * NPAD + base_row)


def _sc_gat_edge(ids2, elt, ert):
    fn = pl.kernel(
        _gat_edge_body,
        out_type=(jax.ShapeDtypeStruct((EEROWS, 8, D), jnp.float32),
                  jax.ShapeDtypeStruct((NC * NPAD, D), jnp.float32)),
        mesh=_MESH,
        scratch_types=[
            pltpu.VMEM_SHARED((NPAD, D), jnp.float32),
            pltpu.VMEM((2 * CHUNK,), jnp.int32),
            pltpu.VMEM((2 * CHUNK,), jnp.int32),
            pltpu.VMEM((CHUNK,), jnp.int32),
            pltpu.VMEM((CHUNK,), jnp.int32),
            pltpu.VMEM((CHUNK, D), jnp.float32),
            pltpu.VMEM((CHUNK, D), jnp.float32),
            pltpu.VMEM((CHUNK, D), jnp.float32),
            pltpu.VMEM((CHUNK, D), jnp.float32),
            pltpu.VMEM((8, D), jnp.float32),
            pltpu.VMEM((CHUNK, D), jnp.float32),
            pltpu.SemaphoreType.DMA,
            pltpu.SemaphoreType.DMA,
            pltpu.SemaphoreType.DMA,
            pltpu.SemaphoreType.DMA,
            pltpu.SemaphoreType.DMA,
            pltpu.SemaphoreType.DMA,
        ],
    )
    return fn(ids2, elt, ert)


def _gat_agg_body(ids_hbm, ee_hbm, feat_hbm, out_hbm, acc_sh,
                  ids_v0, ids_v1, ee_v0, ee_v1, idx_v0, idx_v1,
                  dst_v0, dst_v1, rows_v0, rows_v1, zeros_v,
                  semL0, semL1, semG0, semG1):
    c = lax.axis_index("c")
    s = lax.axis_index("s")
    base_row = s * ROWS_PER_SUB
    _zero_rows(zeros_v, D)

    ids_b = (ids_v0, ids_v1)
    ee_b = (ee_v0, ee_v1)
    idx_b = (idx_v0, idx_v1)
    dst_b = (dst_v0, dst_v1)
    rows_b = (rows_v0, rows_v1)
    semL = (semL0, semL1)
    semG = (semG0, semG1)

    for k in range(HPC):
        head = c * HPC + k
        _zero_slice(zeros_v, acc_sh, base_row)
        plsc.subcore_barrier()
        lane = jnp.broadcast_to(head, (L,))
        off = head * N

        def issue_loads(cid_expr, b):
            pltpu.async_copy(ids_hbm.at[cid_expr], ids_b[b], semL[b])
            pltpu.async_copy(ee_hbm.at[pl.ds(cid_expr * 2, 2)], ee_b[b],
                             semL[b])

        def wait_loads(cid_expr, b):
            pltpu.make_async_copy(ids_hbm.at[cid_expr], ids_b[b],
                                  semL[b]).wait()
            pltpu.make_async_copy(ee_hbm.at[pl.ds(cid_expr * 2, 2)], ee_b[b],
                                  semL[b]).wait()

        def build_and_gather(b):
            for q in range(CH2 // L):
                sl = pl.ds(q * L, L)
                idx_b[b][sl] = ids_b[b][sl] + off
                dst_b[b][sl] = ids_b[b][pl.ds(CH2 + q * L, L)]
            pltpu.async_copy(feat_hbm.at[idx_b[b]], rows_b[b], semG[b])

        issue_loads(s, 0)
        wait_loads(s, 0)
        build_and_gather(0)
        issue_loads(s + NS, 1)

        def trip(t_expr, b):
            nb = 1 - b
            cid = s + t_expr * NS
            wait_loads(cid + NS, nb)
            build_and_gather(nb)
            pltpu.make_async_copy(feat_hbm.at[idx_b[b]], rows_b[b],
                                  semG[b]).wait()

            @pl.loop(0, CH2, step=8)
            def _(jo):
                half = jo >> 6
                g = (jo >> 3) & 7
                for t in range(8):
                    wv = _take16(ee_b[b][half, g, pl.ds(t * L, L)], lane)
                    for tt in range(D // L):
                        sl = pl.ds(tt * L, L)
                        rows_b[b][jo + t, sl] = rows_b[b][jo + t, sl] * wv

            pltpu.sync_copy(rows_b[b], acc_sh.at[dst_b[b]], add=True)
            issue_loads(cid + 2 * NS, b)

        @pl.loop(0, TRIPS_S // 2)
        def _(i2):
            trip(i2 * 2, 0)
            trip(i2 * 2 + 1, 1)

        pltpu.make_async_copy(feat_hbm.at[idx_v0], rows_v0, semG0).wait()
        wait_loads(s + (TRIPS_S + 1) * NS, 1)

        plsc.subcore_barrier()
        _copy_out(acc_sh, out_hbm, base_row, head * NPAD + base_row)
        plsc.subcore_barrier()


def _sc_gat_agg(ids2, ee, feat):
    fn = pl.kernel(
        _gat_agg_body,
        out_type=jax.ShapeDtypeStruct((H * NPAD, D), jnp.float32),
        mesh=_MESH,
        scratch_types=[
            pltpu.VMEM_SHARED((NPAD, D), jnp.float32),
            pltpu.VMEM((2 * CH2,), jnp.int32),
            pltpu.VMEM((2 * CH2,), jnp.int32),
            pltpu.VMEM((2, 8, D), jnp.float32),
            pltpu.VMEM((2, 8, D), jnp.float32),
            pltpu.VMEM((CH2,), jnp.int32),
            pltpu.VMEM((CH2,), jnp.int32),
            pltpu.VMEM((CH2,), jnp.int32),
            pltpu.VMEM((CH2,), jnp.int32),
            pltpu.VMEM((CH2, D), jnp.float32),
            pltpu.VMEM((CH2, D), jnp.float32),
            pltpu.VMEM((CHUNK, D), jnp.float32),
            pltpu.SemaphoreType.DMA,
            pltpu.SemaphoreType.DMA,
            pltpu.SemaphoreType.DMA,
            pltpu.SemaphoreType.DMA,
        ],
    )
    return fn(ids2, ee, feat)


def _gin_mlp_kernel(h_ref, a0_ref, a1_ref, d0_ref, d1_ref, eps_ref,
                    W1_ref, b1_ref, g_ref, beta_ref, W2_ref, b2_ref, o_ref):
    deg = jnp.maximum(d0_ref[:, 0:1] + d1_ref[:, 0:1], 1.0)
    agg = (a0_ref[...] + a1_ref[...]) / deg
    z = (1.0 + eps_ref[0, 0]) * h_ref[...] + agg
    z = jnp.maximum(jnp.dot(z, W1_ref[...],
                            preferred_element_type=jnp.float32) + b1_ref[...],
                    0.0)
    mu = jnp.mean(z, axis=0, keepdims=True)
    var = jnp.mean((z - mu) ** 2, axis=0, keepdims=True)
    z = (z - mu) / jnp.sqrt(var + 1e-5) * g_ref[...] + beta_ref[...]
    z = jnp.dot(z, W2_ref[...], preferred_element_type=jnp.float32) + b2_ref[...]
    o_ref[...] = jnp.maximum(z, 0.0)


def _tc_gin_mlp(h, a0, a1, d0, d1, eps, W1, b1, g, beta, W2, b2):
    return pl.pallas_call(
        _gin_mlp_kernel,
        out_shape=jax.ShapeDtypeStruct((N, HID), jnp.float32),
    )(h, a0, a1, d0, d1, eps.reshape(1, 1), W1, b1.reshape(1, HID),
      g.reshape(1, HID), beta.reshape(1, HID), W2, b2.reshape(1, HID))


def _feat_kernel(h_ref, W_ref, o_ref):
    o_ref[...] = jnp.dot(h_ref[...], W_ref[...],
                         preferred_element_type=jnp.float32)


def _tc_feat(h2, gat_W):
    return pl.pallas_call(
        _feat_kernel,
        grid=(H,),
        in_specs=[
            pl.BlockSpec((N, HID), lambda h: (0, 0)),
            pl.BlockSpec((HID, DH), lambda h: (0, h)),
        ],
        out_specs=pl.BlockSpec((N, DH), lambda h: (h, 0)),
        out_shape=jax.ShapeDtypeStruct((H * N, DH), jnp.float32),
    )(h2, gat_W)


def _tables_kernel(h2_ref, W_ref, al_ref, ar_ref, elt_ref, ert_ref):
    cols_l = []
    cols_r = []
    for h in range(H):
        Wh = W_ref[:, h * DH:(h + 1) * DH]
        cols_l.append(jnp.dot(Wh, al_ref[h, :][:, None],
                              preferred_element_type=jnp.float32))
        cols_r.append(jnp.dot(Wh, ar_ref[h, :][:, None],
                              preferred_element_type=jnp.float32))
    Wal = jnp.concatenate(cols_l, axis=1)
    War = jnp.concatenate(cols_r, axis=1)
    el = jnp.dot(h2_ref[...], Wal, preferred_element_type=jnp.float32)
    er = jnp.dot(h2_ref[...], War, preferred_element_type=jnp.float32)
    elmax = jnp.max(el, axis=0, keepdims=True)
    t = elmax + er
    shift = jnp.where(t >= 0, t, 0.2 * t)
    zero = jnp.zeros((NPAD, D - L), jnp.float32)
    zero16 = jnp.zeros((NPAD - N, L), jnp.float32)
    elt_ref[:N, :H] = el
    elt_ref[:N, H:L] = el
    elt_ref[N:, :L] = zero16
    elt_ref[:, L:] = zero
    ert_ref[:N, :H] = er
    ert_ref[:N, H:L] = shift
    ert_ref[N:, :L] = zero16
    ert_ref[:, L:] = zero


def _tc_tables(h2, gat_W, gat_al, gat_ar):
    return pl.pallas_call(
        _tables_kernel,
        out_shape=(jax.ShapeDtypeStruct((NPAD, D), jnp.float32),
                   jax.ShapeDtypeStruct((NPAD, D), jnp.float32)),
    )(h2, gat_W, gat_al, gat_ar)


def _final_kernel(h1_ref, h2_ref, agg_ref, z0_ref, z1_ref, b_ref, o_ref):
    h = pl.program_id(0)

    @pl.when(h == 0)
    def _():
        o_ref[:, :D] = h1_ref[...]
        o_ref[:, D:2 * D] = h2_ref[...]
        o_ref[:, 2 * D:] = jnp.zeros((N, D), jnp.float32)

    zc = (z0_ref[0, 0, :N] + z1_ref[0, 0, :N]).reshape(N, 1)
    denom = jnp.where(zc > 0, zc, 1.0)
    bias = b_ref[0, 0, :].reshape(1, DH)
    contrib = jnp.maximum(agg_ref[:N, :] / denom + bias, 0.0)
    o_ref[:, 2 * D:] += contrib * (1.0 / H)


def _tc_final(h1, h2, aggout, z0t, z1t, gat_b):
    return pl.pallas_call(
        _final_kernel,
        grid=(H,),
        in_specs=[
            pl.BlockSpec((N, D), lambda h: (0, 0)),
            pl.BlockSpec((N, D), lambda h: (0, 0)),
            pl.BlockSpec((NPAD, D), lambda h: (h, 0)),
            pl.BlockSpec((1, 1, NPAD), lambda h: (h, 0, 0)),
            pl.BlockSpec((1, 1, NPAD), lambda h: (h, 0, 0)),
            pl.BlockSpec((1, 1, DH), lambda h: (h, 0, 0)),
        ],
        out_specs=pl.BlockSpec((N, 3 * D), lambda h: (0, 0)),
        out_shape=jax.ShapeDtypeStruct((N, 3 * D), jnp.float32),
    )(h1, h2, aggout, z0t, z1t, gat_b)


def kernel(x, edge_index, gin1_W1, gin1_b1, gin1_g, gin1_beta, gin1_W2,
           gin1_b2, gin1_eps, gin2_W1, gin2_b1, gin2_g, gin2_beta, gin2_W2,
           gin2_b2, gin2_eps, gat_W, gat_al, gat_ar, gat_b):
    src = edge_index[0]
    dst = edge_index[1]
    npadE = EPAD - E
    srcp = jnp.concatenate([src, jnp.zeros((npadE,), jnp.int32)])
    dstp = jnp.concatenate([dst, jnp.full((npadE,), N, jnp.int32)])
    idsrow = jnp.concatenate([srcp.reshape(TCHP, CHUNK),
                              dstp.reshape(TCHP, CHUNK)], axis=1)
    ids2 = jnp.concatenate(
        [idsrow, jnp.zeros((TCHX - TCHP, 2 * CHUNK), jnp.int32)])

    srcp2 = jnp.concatenate([src, jnp.zeros((EPAD2 - E,), jnp.int32)])
    dstp2 = jnp.concatenate([dst, jnp.full((EPAD2 - E,), N, jnp.int32)])
    idsrow2 = jnp.concatenate([srcp2.reshape(TCP2, CH2),
                               dstp2.reshape(TCP2, CH2)], axis=1)
    ids128 = jnp.concatenate(
        [idsrow2, jnp.zeros((TCX2 - TCP2, 2 * CH2), jnp.int32)])

    degp = _sc_deg(ids2)
    d0 = degp[:N, :L]
    d1 = degp[NPAD:NPAD + N, :L]

    ssum1 = _sc_gin_agg(ids2, x)
    h1 = _tc_gin_mlp(x, ssum1[:N, :], ssum1[NPAD:NPAD + N, :], d0, d1,
                     gin1_eps, gin1_W1, gin1_b1, gin1_g, gin1_beta,
                     gin1_W2, gin1_b2)

    ssum2 = _sc_gin_agg(ids2, h1)
    h2 = _tc_gin_mlp(h1, ssum2[:N, :], ssum2[NPAD:NPAD + N, :], d0, d1,
                     gin2_eps, gin2_W1, gin2_b1, gin2_g, gin2_beta,
                     gin2_W2, gin2_b2)

    feat = _tc_feat(h2, gat_W)
    elt, ert = _tc_tables(h2, gat_W, gat_al, gat_ar)
    ee, zpart = _sc_gat_edge(ids2, elt, ert)
    aggout = _sc_gat_agg(ids128, ee, feat)

    z0t = zpart[:NPAD, :H].T.reshape(H, 1, NPAD)
    z1t = zpart[NPAD:, :H].T.reshape(H, 1, NPAD)
    bt = gat_b.reshape(H, 1, DH)
    return _tc_final(h1, h2, aggout, z0t, z1t, bt)


# speedup vs baseline: 1.2661x; 1.2661x over previous
import jax
import jax.numpy as jnp
from jax import lax
from jax.experimental import pallas as pl
from jax.experimental.pallas import tpu as pltpu
from jax.experimental.pallas import tpu_sc as plsc

N = 10000
E = 320000
D = 128
HID = 128
H = 8
DH = 128

NC = 2
NS = 16
L = 16
NW = NC * NS
NPAD = 10240
ROWS_PER_SUB = NPAD // NS
CHUNK = 64
TCHP = -(-E // (CHUNK * NW)) * NW
EPAD = TCHP * CHUNK
TCHX = TCHP + 2 * NW
TRIPS_W = TCHP // NW
TRIPS_S = TCHP // NS
HPC = H // NC
EEROWS = TCHX

_MESH = plsc.VectorSubcoreMesh(
    core_axis_name="c", subcore_axis_name="s", num_cores=NC, num_subcores=NS)

_TAKE_DNUMS = lax.GatherDimensionNumbers(
    offset_dims=(), collapsed_slice_dims=(0,), start_index_map=(0,))


def _take16(vec, idx):
    return lax.gather(vec, idx[:, None], _TAKE_DNUMS, (1,),
                      mode=lax.GatherScatterMode.PROMISE_IN_BOUNDS)


def _zero_rows(buf, width):
    z = jnp.zeros((L,), jnp.float32)

    @pl.loop(0, buf.shape[0])
    def _(i):
        for t in range(width // L):
            buf[i, pl.ds(t * L, L)] = z


def _zero_slice(zeros_v, shared, base_row):
    @pl.loop(0, ROWS_PER_SUB, step=CHUNK)
    def _(r):
        pltpu.sync_copy(zeros_v, shared.at[pl.ds(base_row + r, CHUNK)])


def _copy_out(shared, out_hbm, base_row, out_base):
    @pl.loop(0, ROWS_PER_SUB, step=CHUNK)
    def _(r):
        pltpu.sync_copy(shared.at[pl.ds(base_row + r, CHUNK)],
                        out_hbm.at[pl.ds(out_base + r, CHUNK)])


def _dst_half(ids_v, dst_v):
    for q in range(4):
        dst_v[pl.ds(q * L, L)] = ids_v[pl.ds(CHUNK + q * L, L)]


def _src_slice(ids_v):
    return ids_v.at[pl.ds(0, CHUNK)]


def _deg_body(ids_hbm, deg_hbm, acc_sh,
              ids_v0, ids_v1, dst_v0, dst_v1, ones_v, semL0, semL1):
    c = lax.axis_index("c")
    s = lax.axis_index("s")
    base_row = s * ROWS_PER_SUB
    _zero_rows(ones_v, D)
    _zero_slice(ones_v, acc_sh, base_row)
    one0 = jnp.where(lax.iota(jnp.int32, L) == 0,
                     jnp.float32(1.0), jnp.float32(0.0))

    @pl.loop(0, CHUNK)
    def _(i):
        ones_v[i, pl.ds(0, L)] = one0

    plsc.subcore_barrier()
    w = c * NS + s
    ids_b = (ids_v0, ids_v1)
    dst_b = (dst_v0, dst_v1)
    semL = (semL0, semL1)

    pltpu.async_copy(ids_hbm.at[w], ids_v0, semL0)
    pltpu.async_copy(ids_hbm.at[w + NW], ids_v1, semL1)

    @pl.loop(0, TRIPS_W // 2)
    def _(i2):
        for b in range(2):
            t = i2 * 2 + b
            cid = w + t * NW
            pltpu.make_async_copy(ids_hbm.at[cid], ids_b[b], semL[b]).wait()
            _dst_half(ids_b[b], dst_b[b])
            pltpu.async_copy(ids_hbm.at[cid + 2 * NW], ids_b[b], semL[b])
            pltpu.sync_copy(ones_v, acc_sh.at[dst_b[b]], add=True)

    t = TRIPS_W - 1
    cid = w + t * NW
    pltpu.make_async_copy(ids_hbm.at[cid], ids_v0, semL0).wait()
    pltpu.make_async_copy(ids_hbm.at[cid + NW], ids_v1, semL1).wait()
    _dst_half(ids_v0, dst_v0)
    pltpu.sync_copy(ones_v, acc_sh.at[dst_v0], add=True)

    plsc.subcore_barrier()
    _copy_out(acc_sh, deg_hbm, base_row, c * NPAD + base_row)


def _sc_deg(ids2):
    fn = pl.kernel(
        _deg_body,
        out_type=jax.ShapeDtypeStruct((NC * NPAD, D), jnp.float32),
        mesh=_MESH,
        scratch_types=[
            pltpu.VMEM_SHARED((NPAD, D), jnp.float32),
            pltpu.VMEM((2 * CHUNK,), jnp.int32),
            pltpu.VMEM((2 * CHUNK,), jnp.int32),
            pltpu.VMEM((CHUNK,), jnp.int32),
            pltpu.VMEM((CHUNK,), jnp.int32),
            pltpu.VMEM((CHUNK, D), jnp.float32),
            pltpu.SemaphoreType.DMA,
            pltpu.SemaphoreType.DMA,
        ],
    )
    return fn(ids2)


def _gin_agg_body(ids_hbm, h_hbm, ssum_hbm, acc_sh,
                  ids_v0, ids_v1, dst_v0, dst_v1, rows_v0, rows_v1,
                  zeros_v, semL0, semL1, semG0, semG1, semS0, semS1):
    c = lax.axis_index("c")
    s = lax.axis_index("s")
    base_row = s * ROWS_PER_SUB
    _zero_rows(zeros_v, D)
    _zero_slice(zeros_v, acc_sh, base_row)

    w = c * NS + s
    ids_b = (ids_v0, ids_v1)
    dst_b = (dst_v0, dst_v1)
    rows_b = (rows_v0, rows_v1)
    semL = (semL0, semL1)
    semG = (semG0, semG1)
    semS = (semS0, semS1)

    pltpu.async_copy(h_hbm.at[pl.ds(0, CHUNK)], rows_v1, semS1)
    plsc.subcore_barrier()

    pltpu.async_copy(ids_hbm.at[w], ids_v0, semL0).wait()
    pltpu.async_copy(h_hbm.at[_src_slice(ids_v0)], rows_v0, semG0)
    pltpu.async_copy(ids_hbm.at[w + NW], ids_v1, semL1)

    def trip(t_expr, b):
        nb = 1 - b
        cid = w + t_expr * NW
        pltpu.make_async_copy(ids_hbm.at[cid + NW], ids_b[nb],
                              semL[nb]).wait()
        pltpu.make_async_copy(rows_b[nb], acc_sh.at[dst_b[nb]],
                              semS[nb]).wait()
        pltpu.async_copy(h_hbm.at[_src_slice(ids_b[nb])], rows_b[nb],
                         semG[nb])
        pltpu.make_async_copy(h_hbm.at[_src_slice(ids_b[b])], rows_b[b],
                              semG[b]).wait()
        _dst_half(ids_b[b], dst_b[b])
        pltpu.async_copy(rows_b[b], acc_sh.at[dst_b[b]], semS[b], add=True)
        pltpu.async_copy(ids_hbm.at[cid + 2 * NW], ids_b[b], semL[b])

    @pl.loop(0, TRIPS_W // 2)
    def _(i2):
        trip(i2 * 2, 0)
        trip(i2 * 2 + 1, 1)

    t = TRIPS_W - 1
    cid = w + t * NW
    pltpu.make_async_copy(ids_hbm.at[cid + NW], ids_v1, semL1).wait()
    pltpu.make_async_copy(h_hbm.at[_src_slice(ids_v0)], rows_v0,
                          semG0).wait()
    _dst_half(ids_v0, dst_v0)
    pltpu.sync_copy(rows_v0, acc_sh.at[dst_v0], add=True)
    pltpu.make_async_copy(rows_v1, acc_sh.at[dst_v1], semS1).wait()

    plsc.subcore_barrier()
    _copy_out(acc_sh, ssum_hbm, base_row, c * NPAD + base_row)


def _sc_gin_agg(ids2, h):
    fn = pl.kernel(
        _gin_agg_body,
        out_type=jax.ShapeDtypeStruct((NC * NPAD, D), jnp.float32),
        mesh=_MESH,
        scratch_types=[
            pltpu.VMEM_SHARED((NPAD, D), jnp.float32),
            pltpu.VMEM((2 * CHUNK,), jnp.int32),
            pltpu.VMEM((2 * CHUNK,), jnp.int32),
            pltpu.VMEM((CHUNK,), jnp.int32),
            pltpu.VMEM((CHUNK,), jnp.int32),
            pltpu.VMEM((CHUNK, D), jnp.float32),
            pltpu.VMEM((CHUNK, D), jnp.float32),
            pltpu.VMEM((CHUNK, D), jnp.float32),
            pltpu.SemaphoreType.DMA,
            pltpu.SemaphoreType.DMA,
            pltpu.SemaphoreType.DMA,
            pltpu.SemaphoreType.DMA,
            pltpu.SemaphoreType.DMA,
            pltpu.SemaphoreType.DMA,
        ],
    )
    return fn(ids2, h)


def _gat_edge_body(ids_hbm, elt_hbm, ert_hbm, ee_hbm, z_hbm, z_sh,
                   ids_v0, ids_v1, dst_v0, dst_v1, s_v0, s_v1, d_v0, d_v1,
                   ee_v, zrow_v,
                   semL0, semL1, semG0, semG1, semH0, semH1):
    c = lax.axis_index("c")
    s = lax.axis_index("s")
    base_row = s * ROWS_PER_SUB
    _zero_rows(zrow_v, D)
    _zero_slice(zrow_v, z_sh, base_row)
    plsc.subcore_barrier()
    rot = (lax.iota(jnp.int32, L) + 8) & 15
    w = c * NS + s

    ids_b = (ids_v0, ids_v1)
    dst_b = (dst_v0, dst_v1)
    s_b = (s_v0, s_v1)
    d_b = (d_v0, d_v1)
    semL = (semL0, semL1)
    semG = (semG0, semG1)
    semH = (semH0, semH1)

    def issue_gathers(b):
        pltpu.async_copy(elt_hbm.at[_src_slice(ids_b[b])], s_b[b], semG[b])
        _dst_half(ids_b[b], dst_b[b])
        pltpu.async_copy(ert_hbm.at[dst_b[b]], d_b[b], semH[b])

    pltpu.async_copy(ids_hbm.at[w], ids_v0, semL0).wait()
    issue_gathers(0)
    pltpu.async_copy(ids_hbm.at[w + NW], ids_v1, semL1)

    def trip(t_expr, b):
        nb = 1 - b
        cid = w + t_expr * NW
        pltpu.make_async_copy(ids_hbm.at[cid + NW], ids_b[nb],
                              semL[nb]).wait()
        issue_gathers(nb)
        pltpu.make_async_copy(elt_hbm.at[_src_slice(ids_b[b])], s_b[b],
                              semG[b]).wait()
        pltpu.make_async_copy(ert_hbm.at[dst_b[b]], d_b[b], semH[b]).wait()

        @pl.loop(0, CHUNK, step=8)
        def _(jo):
            g = jo >> 3
            for t in range(8):
                srow = s_b[b][jo + t, pl.ds(0, L)]
                drow = d_b[b][jo + t, pl.ds(0, L)]
                esum = srow + drow
                e = jnp.where(esum >= 0, esum, 0.2 * esum)
                shift = _take16(drow, rot)
                ee = jnp.exp(e - shift)
                zrow_v[jo + t, pl.ds(0, L)] = ee
                ee_v[g, pl.ds(t * L, L)] = ee

        pltpu.sync_copy(ee_v, ee_hbm.at[cid])
        pltpu.sync_copy(zrow_v, z_sh.at[dst_b[b]], add=True)
        pltpu.async_copy(ids_hbm.at[cid + 2 * NW], ids_b[b], semL[b])

    @pl.loop(0, TRIPS_W // 2)
    def _(i2):
        trip(i2 * 2, 0)
        trip(i2 * 2 + 1, 1)

    t = TRIPS_W - 1
    cid = w + t * NW
    pltpu.make_async_copy(ids_hbm.at[cid + NW], ids_v1, semL1).wait()
    pltpu.make_async_copy(elt_hbm.at[_src_slice(ids_v0)], s_v0,
                          semG0).wait()
    pltpu.make_async_copy(ert_hbm.at[dst_v0], d_v0, semH0).wait()

    @pl.loop(0, CHUNK, step=8)
    def _(jo):
        g = jo >> 3
        for tt in range(8):
            srow = s_v0[jo + tt, pl.ds(0, L)]
            drow = d_v0[jo + tt, pl.ds(0, L)]
            esum = srow + drow
            e = jnp.where(esum >= 0, esum, 0.2 * esum)
            shift = _take16(drow, rot)
            ee = jnp.exp(e - shift)
            zrow_v[jo + tt, pl.ds(0, L)] = ee
            ee_v[g, pl.ds(tt * L, L)] = ee

    pltpu.sync_copy(ee_v, ee_hbm.at[cid])
    pltpu.sync_copy(zrow_v, z_sh.at[dst_v0], add=True)

    plsc.subcore_barrier()
    _copy_out(z_sh, z_hbm, base_row, c * NPAD + base_row)


def _sc_gat_edge(ids2, elt, ert):
    fn = pl.kernel(
        _gat_edge_body,
        out_type=(jax.ShapeDtypeStruct((EEROWS, 8, D), jnp.float32),
                  jax.ShapeDtypeStruct((NC * NPAD, D), jnp.float32)),
        mesh=_MESH,
        scratch_types=[
            pltpu.VMEM_SHARED((NPAD, D), jnp.float32),
            pltpu.VMEM((2 * CHUNK,), jnp.int32),
            pltpu.VMEM((2 * CHUNK,), jnp.int32),
            pltpu.VMEM((CHUNK,), jnp.int32),
            pltpu.VMEM((CHUNK,), jnp.int32),
            pltpu.VMEM((CHUNK, D), jnp.float32),
            pltpu.VMEM((CHUNK, D), jnp.float32),
            pltpu.VMEM((CHUNK, D), jnp.float32),
            pltpu.VMEM((CHUNK, D), jnp.float32),
            pltpu.VMEM((8, D), jnp.float32),
            pltpu.VMEM((CHUNK, D), jnp.float32),
            pltpu.SemaphoreType.DMA,
            pltpu.SemaphoreType.DMA,
            pltpu.SemaphoreType.DMA,
            pltpu.SemaphoreType.DMA,
            pltpu.SemaphoreType.DMA,
            pltpu.SemaphoreType.DMA,
        ],
    )
    return fn(ids2, elt, ert)


def _gat_agg_body(ids_hbm, ee_hbm, feat_hbm, out_hbm, acc_sh,
                  ids_v0, ids_v1, ee_v0, ee_v1, idx_v0, idx_v1,
                  dst_v0, dst_v1, rows_v0, rows_v1, zeros_v,
                  semL0, semL1, semG0, semG1, semS0, semS1):
    c = lax.axis_index("c")
    s = lax.axis_index("s")
    base_row = s * ROWS_PER_SUB
    _zero_rows(zeros_v, D)

    ids_b = (ids_v0, ids_v1)
    ee_b = (ee_v0, ee_v1)
    idx_b = (idx_v0, idx_v1)
    dst_b = (dst_v0, dst_v1)
    rows_b = (rows_v0, rows_v1)
    semL = (semL0, semL1)
    semG = (semG0, semG1)
    semS = (semS0, semS1)

    for k in range(HPC):
        head = c * HPC + k
        _zero_slice(zeros_v, acc_sh, base_row)
        pltpu.async_copy(feat_hbm.at[pl.ds(0, CHUNK)], rows_v1, semS1)
        plsc.subcore_barrier()
        lane = jnp.broadcast_to(head, (L,))
        off = head * N

        def issue_loads(cid_expr, b):
            pltpu.async_copy(ids_hbm.at[cid_expr], ids_b[b], semL[b])
            pltpu.async_copy(ee_hbm.at[cid_expr], ee_b[b], semL[b])

        def wait_loads(cid_expr, b):
            pltpu.make_async_copy(ids_hbm.at[cid_expr], ids_b[b],
                                  semL[b]).wait()
            pltpu.make_async_copy(ee_hbm.at[cid_expr], ee_b[b],
                                  semL[b]).wait()

        def build_and_gather(b):
            for q in range(CHUNK // L):
                sl = pl.ds(q * L, L)
                idx_b[b][sl] = ids_b[b][sl] + off
                dst_b[b][sl] = ids_b[b][pl.ds(CHUNK + q * L, L)]
            pltpu.async_copy(feat_hbm.at[idx_b[b]], rows_b[b], semG[b])

        issue_loads(s, 0)
        wait_loads(s, 0)
        build_and_gather(0)
        issue_loads(s + NS, 1)

        def trip(t_expr, b):
            nb = 1 - b
            cid = s + t_expr * NS
            wait_loads(cid + NS, nb)
            pltpu.make_async_copy(rows_b[nb], acc_sh.at[dst_b[nb]],
                                  semS[nb]).wait()
            build_and_gather(nb)
            pltpu.make_async_copy(feat_hbm.at[idx_b[b]], rows_b[b],
                                  semG[b]).wait()

            @pl.loop(0, CHUNK, step=8)
            def _(jo):
                g = jo >> 3
                for t in range(8):
                    wv = _take16(ee_b[b][g, pl.ds(t * L, L)], lane)
                    for tt in range(D // L):
                        sl = pl.ds(tt * L, L)
                        rows_b[b][jo + t, sl] = rows_b[b][jo + t, sl] * wv

            pltpu.async_copy(rows_b[b], acc_sh.at[dst_b[b]], semS[b],
                             add=True)
            issue_loads(cid + 2 * NS, b)

        @pl.loop(0, TRIPS_S // 2)
        def _(i2):
            trip(i2 * 2, 0)
            trip(i2 * 2 + 1, 1)

        pltpu.make_async_copy(feat_hbm.at[idx_v0], rows_v0, semG0).wait()
        wait_loads(s + (TRIPS_S + 1) * NS, 1)
        pltpu.make_async_copy(rows_v1, acc_sh.at[dst_v1], semS1).wait()

        plsc.subcore_barrier()
        _copy_out(acc_sh, out_hbm, base_row, head * NPAD + base_row)
        plsc.subcore_barrier()


def _sc_gat_agg(ids2, ee, feat):
    fn = pl.kernel(
        _gat_agg_body,
        out_type=jax.ShapeDtypeStruct((H * NPAD, D), jnp.float32),
        mesh=_MESH,
        scratch_types=[
            pltpu.VMEM_SHARED((NPAD, D), jnp.float32),
            pltpu.VMEM((2 * CHUNK,), jnp.int32),
            pltpu.VMEM((2 * CHUNK,), jnp.int32),
            pltpu.VMEM((8, D), jnp.float32),
            pltpu.VMEM((8, D), jnp.float32),
            pltpu.VMEM((CHUNK,), jnp.int32),
            pltpu.VMEM((CHUNK,), jnp.int32),
            pltpu.VMEM((CHUNK,), jnp.int32),
            pltpu.VMEM((CHUNK,), jnp.int32),
            pltpu.VMEM((CHUNK, D), jnp.float32),
            pltpu.VMEM((CHUNK, D), jnp.float32),
            pltpu.VMEM((CHUNK, D), jnp.float32),
            pltpu.SemaphoreType.DMA,
            pltpu.SemaphoreType.DMA,
            pltpu.SemaphoreType.DMA,
            pltpu.SemaphoreType.DMA,
            pltpu.SemaphoreType.DMA,
            pltpu.SemaphoreType.DMA,
        ],
    )
    return fn(ids2, ee, feat)


def _gin_mlp_kernel(h_ref, a0_ref, a1_ref, d0_ref, d1_ref, eps_ref,
                    W1_ref, b1_ref, g_ref, beta_ref, W2_ref, b2_ref, o_ref):
    deg = jnp.maximum(d0_ref[:, 0:1] + d1_ref[:, 0:1], 1.0)
    agg = (a0_ref[...] + a1_ref[...]) / deg
    z = (1.0 + eps_ref[0, 0]) * h_ref[...] + agg
    z = jnp.maximum(jnp.dot(z, W1_ref[...],
                            preferred_element_type=jnp.float32) + b1_ref[...],
                    0.0)
    mu = jnp.mean(z, axis=0, keepdims=True)
    var = jnp.mean((z - mu) ** 2, axis=0, keepdims=True)
    z = (z - mu) / jnp.sqrt(var + 1e-5) * g_ref[...] + beta_ref[...]
    z = jnp.dot(z, W2_ref[...], preferred_element_type=jnp.float32) + b2_ref[...]
    o_ref[...] = jnp.maximum(z, 0.0)


def _tc_gin_mlp(h, a0, a1, d0, d1, eps, W1, b1, g, beta, W2, b2):
    return pl.pallas_call(
        _gin_mlp_kernel,
        out_shape=jax.ShapeDtypeStruct((N, HID), jnp.float32),
    )(h, a0, a1, d0, d1, eps.reshape(1, 1), W1, b1.reshape(1, HID),
      g.reshape(1, HID), beta.reshape(1, HID), W2, b2.reshape(1, HID))


def _feat_kernel(h_ref, W_ref, o_ref):
    o_ref[...] = jnp.dot(h_ref[...], W_ref[...],
                         preferred_element_type=jnp.float32)


def _tc_feat(h2, gat_W):
    return pl.pallas_call(
        _feat_kernel,
        grid=(H,),
        in_specs=[
            pl.BlockSpec((N, HID), lambda h: (0, 0)),
            pl.BlockSpec((HID, DH), lambda h: (0, h)),
        ],
        out_specs=pl.BlockSpec((N, DH), lambda h: (h, 0)),
        out_shape=jax.ShapeDtypeStruct((H * N, DH), jnp.float32),
    )(h2, gat_W)


def _tables_kernel(h2_ref, W_ref, al_ref, ar_ref, elt_ref, ert_ref):
    cols_l = []
    cols_r = []
    for h in range(H):
        Wh = W_ref[:, h * DH:(h + 1) * DH]
        cols_l.append(jnp.dot(Wh, al_ref[h, :][:, None],
                              preferred_element_type=jnp.float32))
        cols_r.append(jnp.dot(Wh, ar_ref[h, :][:, None],
                              preferred_element_type=jnp.float32))
    Wal = jnp.concatenate(cols_l, axis=1)
    War = jnp.concatenate(cols_r, axis=1)
    el = jnp.dot(h2_ref[...], Wal, preferred_element_type=jnp.float32)
    er = jnp.dot(h2_ref[...], War, preferred_element_type=jnp.float32)
    elmax = jnp.max(el, axis=0, keepdims=True)
    t = elmax + er
    shift = jnp.where(t >= 0, t, 0.2 * t)
    zero = jnp.zeros((NPAD, D - L), jnp.float32)
    zero16 = jnp.zeros((NPAD - N, L), jnp.float32)
    elt_ref[:N, :H] = el
    elt_ref[:N, H:L] = el
    elt_ref[N:, :L] = zero16
    elt_ref[:, L:] = zero
    ert_ref[:N, :H] = er
    ert_ref[:N, H:L] = shift
    ert_ref[N:, :L] = zero16
    ert_ref[:, L:] = zero


def _tc_tables(h2, gat_W, gat_al, gat_ar):
    return pl.pallas_call(
        _tables_kernel,
        out_shape=(jax.ShapeDtypeStruct((NPAD, D), jnp.float32),
                   jax.ShapeDtypeStruct((NPAD, D), jnp.float32)),
    )(h2, gat_W, gat_al, gat_ar)


def _final_kernel(h1_ref, h2_ref, agg_ref, z0_ref, z1_ref, b_ref, o_ref):
    h = pl.program_id(0)

    @pl.when(h == 0)
    def _():
        o_ref[:, :D] = h1_ref[...]
        o_ref[:, D:2 * D] = h2_ref[...]
        o_ref[:, 2 * D:] = jnp.zeros((N, D), jnp.float32)

    zc = (z0_ref[0, 0, :N] + z1_ref[0, 0, :N]).reshape(N, 1)
    denom = jnp.where(zc > 0, zc, 1.0)
    bias = b_ref[0, 0, :].reshape(1, DH)
    contrib = jnp.maximum(agg_ref[:N, :] / denom + bias, 0.0)
    o_ref[:, 2 * D:] += contrib * (1.0 / H)


def _tc_final(h1, h2, aggout, z0t, z1t, gat_b):
    return pl.pallas_call(
        _final_kernel,
        grid=(H,),
        in_specs=[
            pl.BlockSpec((N, D), lambda h: (0, 0)),
            pl.BlockSpec((N, D), lambda h: (0, 0)),
            pl.BlockSpec((NPAD, D), lambda h: (h, 0)),
            pl.BlockSpec((1, 1, NPAD), lambda h: (h, 0, 0)),
            pl.BlockSpec((1, 1, NPAD), lambda h: (h, 0, 0)),
            pl.BlockSpec((1, 1, DH), lambda h: (h, 0, 0)),
        ],
        out_specs=pl.BlockSpec((N, 3 * D), lambda h: (0, 0)),
        out_shape=jax.ShapeDtypeStruct((N, 3 * D), jnp.float32),
    )(h1, h2, aggout, z0t, z1t, gat_b)


def kernel(x, edge_index, gin1_W1, gin1_b1, gin1_g, gin1_beta, gin1_W2,
           gin1_b2, gin1_eps, gin2_W1, gin2_b1, gin2_g, gin2_beta, gin2_W2,
           gin2_b2, gin2_eps, gat_W, gat_al, gat_ar, gat_b):
    src = edge_index[0]
    dst = edge_index[1]
    npadE = EPAD - E
    srcp = jnp.concatenate([src, jnp.zeros((npadE,), jnp.int32)])
    dstp = jnp.concatenate([dst, jnp.full((npadE,), N, jnp.int32)])
    idsrow = jnp.concatenate([srcp.reshape(TCHP, CHUNK),
                              dstp.reshape(TCHP, CHUNK)], axis=1)
    ids2 = jnp.concatenate(
        [idsrow, jnp.zeros((TCHX - TCHP, 2 * CHUNK), jnp.int32)])


    degp = _sc_deg(ids2)
    d0 = degp[:N, :L]
    d1 = degp[NPAD:NPAD + N, :L]

    ssum1 = _sc_gin_agg(ids2, x)
    h1 = _tc_gin_mlp(x, ssum1[:N, :], ssum1[NPAD:NPAD + N, :], d0, d1,
                     gin1_eps, gin1_W1, gin1_b1, gin1_g, gin1_beta,
                     gin1_W2, gin1_b2)

    ssum2 = _sc_gin_agg(ids2, h1)
    h2 = _tc_gin_mlp(h1, ssum2[:N, :], ssum2[NPAD:NPAD + N, :], d0, d1,
                     gin2_eps, gin2_W1, gin2_b1, gin2_g, gin2_beta,
                     gin2_W2, gin2_b2)

    feat = _tc_feat(h2, gat_W)
    elt, ert = _tc_tables(h2, gat_W, gat_al, gat_ar)
    ee, zpart = _sc_gat_edge(ids2, elt, ert)
    aggout = _sc_gat_agg(ids2, ee, feat)

    z0t = zpart[:NPAD, :H].T.reshape(H, 1, NPAD)
    z1t = zpart[NPAD:, :H].T.reshape(H, 1, NPAD)
    bt = gat_b.reshape(H, 1, DH)
    return _tc_final(h1, h2, aggout, z0t, z1t, bt)


# speedup vs baseline: 1.2848x; 1.0148x over previous
import jax
import jax.numpy as jnp
from jax import lax
from jax.experimental import pallas as pl
from jax.experimental.pallas import tpu as pltpu
from jax.experimental.pallas import tpu_sc as plsc

N = 10000
E = 320000
D = 128
HID = 128
H = 8
DH = 128

NC = 2
NS = 16
L = 16
NW = NC * NS
NPAD = 10240
ROWS_PER_SUB = NPAD // NS
CHUNK = 64
TCHP = -(-E // (CHUNK * NW)) * NW
EPAD = TCHP * CHUNK
TCHX = TCHP + 2 * NW
TRIPS_W = TCHP // NW
TRIPS_S = TCHP // NS
HPC = H // NC
EEROWS = TCHX

_MESH = plsc.VectorSubcoreMesh(
    core_axis_name="c", subcore_axis_name="s", num_cores=NC, num_subcores=NS)

_TAKE_DNUMS = lax.GatherDimensionNumbers(
    offset_dims=(), collapsed_slice_dims=(0,), start_index_map=(0,))


def _take16(vec, idx):
    return lax.gather(vec, idx[:, None], _TAKE_DNUMS, (1,),
                      mode=lax.GatherScatterMode.PROMISE_IN_BOUNDS)


def _zero_rows(buf, width):
    z = jnp.zeros((L,), jnp.float32)

    @pl.loop(0, buf.shape[0])
    def _(i):
        for t in range(width // L):
            buf[i, pl.ds(t * L, L)] = z


def _zero_slice(zeros_v, shared, base_row):
    @pl.loop(0, ROWS_PER_SUB, step=CHUNK)
    def _(r):
        pltpu.sync_copy(zeros_v, shared.at[pl.ds(base_row + r, CHUNK)])


def _copy_out(shared, out_hbm, base_row, out_base):
    @pl.loop(0, ROWS_PER_SUB, step=CHUNK)
    def _(r):
        pltpu.sync_copy(shared.at[pl.ds(base_row + r, CHUNK)],
                        out_hbm.at[pl.ds(out_base + r, CHUNK)])


def _dst_half(ids_v, dst_v):
    for q in range(4):
        dst_v[pl.ds(q * L, L)] = ids_v[pl.ds(CHUNK + q * L, L)]


def _src_slice(ids_v):
    return ids_v.at[pl.ds(0, CHUNK)]


def _deg_body(ids_hbm, deg_hbm, acc_sh,
              ids_v0, ids_v1, dst_v0, dst_v1, ones_v, semL0, semL1):
    c = lax.axis_index("c")
    s = lax.axis_index("s")
    base_row = s * ROWS_PER_SUB
    _zero_rows(ones_v, D)
    _zero_slice(ones_v, acc_sh, base_row)
    one0 = jnp.where(lax.iota(jnp.int32, L) == 0,
                     jnp.float32(1.0), jnp.float32(0.0))

    @pl.loop(0, CHUNK)
    def _(i):
        ones_v[i, pl.ds(0, L)] = one0

    plsc.subcore_barrier()
    w = c * NS + s
    ids_b = (ids_v0, ids_v1)
    dst_b = (dst_v0, dst_v1)
    semL = (semL0, semL1)

    pltpu.async_copy(ids_hbm.at[w], ids_v0, semL0)
    pltpu.async_copy(ids_hbm.at[w + NW], ids_v1, semL1)

    @pl.loop(0, TRIPS_W // 2)
    def _(i2):
        for b in range(2):
            t = i2 * 2 + b
            cid = w + t * NW
            pltpu.make_async_copy(ids_hbm.at[cid], ids_b[b], semL[b]).wait()
            _dst_half(ids_b[b], dst_b[b])
            pltpu.async_copy(ids_hbm.at[cid + 2 * NW], ids_b[b], semL[b])
            pltpu.sync_copy(ones_v, acc_sh.at[dst_b[b]], add=True)

    t = TRIPS_W - 1
    cid = w + t * NW
    pltpu.make_async_copy(ids_hbm.at[cid], ids_v0, semL0).wait()
    pltpu.make_async_copy(ids_hbm.at[cid + NW], ids_v1, semL1).wait()
    _dst_half(ids_v0, dst_v0)
    pltpu.sync_copy(ones_v, acc_sh.at[dst_v0], add=True)

    plsc.subcore_barrier()
    _copy_out(acc_sh, deg_hbm, base_row, c * NPAD + base_row)


def _sc_deg(ids2):
    fn = pl.kernel(
        _deg_body,
        out_type=jax.ShapeDtypeStruct((NC * NPAD, D), jnp.float32),
        mesh=_MESH,
        scratch_types=[
            pltpu.VMEM_SHARED((NPAD, D), jnp.float32),
            pltpu.VMEM((2 * CHUNK,), jnp.int32),
            pltpu.VMEM((2 * CHUNK,), jnp.int32),
            pltpu.VMEM((CHUNK,), jnp.int32),
            pltpu.VMEM((CHUNK,), jnp.int32),
            pltpu.VMEM((CHUNK, D), jnp.float32),
            pltpu.SemaphoreType.DMA,
            pltpu.SemaphoreType.DMA,
        ],
    )
    return fn(ids2)


def _gin_agg_body(ids_hbm, h_hbm, ssum_hbm, acc_sh,
                  ids_v0, ids_v1, dst_v0, dst_v1, rows_v0, rows_v1,
                  zeros_v, semL0, semL1, semG0, semG1, semS0, semS1):
    c = lax.axis_index("c")
    s = lax.axis_index("s")
    base_row = s * ROWS_PER_SUB
    _zero_rows(zeros_v, D)
    _zero_slice(zeros_v, acc_sh, base_row)

    w = c * NS + s
    ids_b = (ids_v0, ids_v1)
    dst_b = (dst_v0, dst_v1)
    rows_b = (rows_v0, rows_v1)
    semL = (semL0, semL1)
    semG = (semG0, semG1)
    semS = (semS0, semS1)

    pltpu.async_copy(h_hbm.at[pl.ds(0, CHUNK)], rows_v1, semS1)
    plsc.subcore_barrier()

    pltpu.async_copy(ids_hbm.at[w], ids_v0, semL0).wait()
    pltpu.async_copy(h_hbm.at[_src_slice(ids_v0)], rows_v0, semG0)
    pltpu.async_copy(ids_hbm.at[w + NW], ids_v1, semL1)

    def trip(t_expr, b):
        nb = 1 - b
        cid = w + t_expr * NW
        pltpu.make_async_copy(ids_hbm.at[cid + NW], ids_b[nb],
                              semL[nb]).wait()
        pltpu.make_async_copy(rows_b[nb], acc_sh.at[dst_b[nb]],
                              semS[nb]).wait()
        pltpu.async_copy(h_hbm.at[_src_slice(ids_b[nb])], rows_b[nb],
                         semG[nb])
        pltpu.make_async_copy(h_hbm.at[_src_slice(ids_b[b])], rows_b[b],
                              semG[b]).wait()
        _dst_half(ids_b[b], dst_b[b])
        pltpu.async_copy(rows_b[b], acc_sh.at[dst_b[b]], semS[b], add=True)
        pltpu.async_copy(ids_hbm.at[cid + 2 * NW], ids_b[b], semL[b])

    @pl.loop(0, TRIPS_W // 2)
    def _(i2):
        trip(i2 * 2, 0)
        trip(i2 * 2 + 1, 1)

    t = TRIPS_W - 1
    cid = w + t * NW
    pltpu.make_async_copy(ids_hbm.at[cid + NW], ids_v1, semL1).wait()
    pltpu.make_async_copy(h_hbm.at[_src_slice(ids_v0)], rows_v0,
                          semG0).wait()
    _dst_half(ids_v0, dst_v0)
    pltpu.sync_copy(rows_v0, acc_sh.at[dst_v0], add=True)
    pltpu.make_async_copy(rows_v1, acc_sh.at[dst_v1], semS1).wait()

    plsc.subcore_barrier()
    _copy_out(acc_sh, ssum_hbm, base_row, c * NPAD + base_row)


def _sc_gin_agg(ids2, h):
    fn = pl.kernel(
        _gin_agg_body,
        out_type=jax.ShapeDtypeStruct((NC * NPAD, D), jnp.float32),
        mesh=_MESH,
        scratch_types=[
            pltpu.VMEM_SHARED((NPAD, D), jnp.float32),
            pltpu.VMEM((2 * CHUNK,), jnp.int32),
            pltpu.VMEM((2 * CHUNK,), jnp.int32),
            pltpu.VMEM((CHUNK,), jnp.int32),
            pltpu.VMEM((CHUNK,), jnp.int32),
            pltpu.VMEM((CHUNK, D), jnp.float32),
            pltpu.VMEM((CHUNK, D), jnp.float32),
            pltpu.VMEM((CHUNK, D), jnp.float32),
            pltpu.SemaphoreType.DMA,
            pltpu.SemaphoreType.DMA,
            pltpu.SemaphoreType.DMA,
            pltpu.SemaphoreType.DMA,
            pltpu.SemaphoreType.DMA,
            pltpu.SemaphoreType.DMA,
        ],
    )
    return fn(ids2, h)


def _gat_edge_body(ids_hbm, elt_hbm, ert_hbm, ee_hbm, z_hbm, z_sh,
                   ids_v0, ids_v1, dst_v0, dst_v1, s_v0, s_v1, d_v0, d_v1,
                   ee_v0, ee_v1, zrow_v,
                   semL0, semL1, semG0, semG1, semH0, semH1, semE0, semE1):
    c = lax.axis_index("c")
    s = lax.axis_index("s")
    base_row = s * ROWS_PER_SUB
    _zero_rows(zrow_v, D)
    _zero_slice(zrow_v, z_sh, base_row)
    plsc.subcore_barrier()
    rot = (lax.iota(jnp.int32, L) + 8) & 15
    w = c * NS + s

    ids_b = (ids_v0, ids_v1)
    dst_b = (dst_v0, dst_v1)
    s_b = (s_v0, s_v1)
    d_b = (d_v0, d_v1)
    ee_b = (ee_v0, ee_v1)
    semL = (semL0, semL1)
    semG = (semG0, semG1)
    semH = (semH0, semH1)
    semE = (semE0, semE1)
    pltpu.async_copy(ee_hbm.at[0], ee_v0, semE0)
    pltpu.async_copy(ee_hbm.at[1], ee_v1, semE1)

    def issue_gathers(b):
        pltpu.async_copy(elt_hbm.at[_src_slice(ids_b[b])], s_b[b], semG[b])
        _dst_half(ids_b[b], dst_b[b])
        pltpu.async_copy(ert_hbm.at[dst_b[b]], d_b[b], semH[b])

    pltpu.async_copy(ids_hbm.at[w], ids_v0, semL0).wait()
    issue_gathers(0)
    pltpu.async_copy(ids_hbm.at[w + NW], ids_v1, semL1)

    def trip(t_expr, b):
        nb = 1 - b
        cid = w + t_expr * NW
        pltpu.make_async_copy(ids_hbm.at[cid + NW], ids_b[nb],
                              semL[nb]).wait()
        issue_gathers(nb)
        pltpu.make_async_copy(elt_hbm.at[_src_slice(ids_b[b])], s_b[b],
                              semG[b]).wait()
        pltpu.make_async_copy(ert_hbm.at[dst_b[b]], d_b[b], semH[b]).wait()
        pltpu.make_async_copy(ee_b[b], ee_hbm.at[cid], semE[b]).wait()

        @pl.loop(0, CHUNK, step=8)
        def _(jo):
            g = jo >> 3
            for t in range(8):
                srow = s_b[b][jo + t, pl.ds(0, L)]
                drow = d_b[b][jo + t, pl.ds(0, L)]
                esum = srow + drow
                e = jnp.where(esum >= 0, esum, 0.2 * esum)
                shift = _take16(drow, rot)
                ee = jnp.exp(e - shift)
                zrow_v[jo + t, pl.ds(0, L)] = ee
                ee_b[b][g, pl.ds(t * L, L)] = ee

        pltpu.async_copy(ee_b[b], ee_hbm.at[cid], semE[b])
        pltpu.sync_copy(zrow_v, z_sh.at[dst_b[b]], add=True)
        pltpu.async_copy(ids_hbm.at[cid + 2 * NW], ids_b[b], semL[b])

    @pl.loop(0, TRIPS_W // 2)
    def _(i2):
        trip(i2 * 2, 0)
        trip(i2 * 2 + 1, 1)

    t = TRIPS_W - 1
    cid = w + t * NW
    pltpu.make_async_copy(ids_hbm.at[cid + NW], ids_v1, semL1).wait()
    pltpu.make_async_copy(elt_hbm.at[_src_slice(ids_v0)], s_v0,
                          semG0).wait()
    pltpu.make_async_copy(ert_hbm.at[dst_v0], d_v0, semH0).wait()
    pltpu.make_async_copy(ee_v0, ee_hbm.at[cid], semE0).wait()

    @pl.loop(0, CHUNK, step=8)
    def _(jo):
        g = jo >> 3
        for tt in range(8):
            srow = s_v0[jo + tt, pl.ds(0, L)]
            drow = d_v0[jo + tt, pl.ds(0, L)]
            esum = srow + drow
            e = jnp.where(esum >= 0, esum, 0.2 * esum)
            shift = _take16(drow, rot)
            ee = jnp.exp(e - shift)
            zrow_v[jo + tt, pl.ds(0, L)] = ee
            ee_v0[g, pl.ds(tt * L, L)] = ee

    pltpu.sync_copy(ee_v0, ee_hbm.at[cid])
    pltpu.sync_copy(zrow_v, z_sh.at[dst_v0], add=True)
    pltpu.make_async_copy(ee_v1, ee_hbm.at[cid], semE1).wait()

    plsc.subcore_barrier()
    _copy_out(z_sh, z_hbm, base_row, c * NPAD + base_row)


def _sc_gat_edge(ids2, elt, ert):
    fn = pl.kernel(
        _gat_edge_body,
        out_type=(jax.ShapeDtypeStruct((EEROWS, 8, D), jnp.float32),
                  jax.ShapeDtypeStruct((NC * NPAD, D), jnp.float32)),
        mesh=_MESH,
        scratch_types=[
            pltpu.VMEM_SHARED((NPAD, D), jnp.float32),
            pltpu.VMEM((2 * CHUNK,), jnp.int32),
            pltpu.VMEM((2 * CHUNK,), jnp.int32),
            pltpu.VMEM((CHUNK,), jnp.int32),
            pltpu.VMEM((CHUNK,), jnp.int32),
            pltpu.VMEM((CHUNK, D), jnp.float32),
            pltpu.VMEM((CHUNK, D), jnp.float32),
            pltpu.VMEM((CHUNK, D), jnp.float32),
            pltpu.VMEM((CHUNK, D), jnp.float32),
            pltpu.VMEM((8, D), jnp.float32),
            pltpu.VMEM((8, D), jnp.float32),
            pltpu.VMEM((CHUNK, D), jnp.float32),
            pltpu.SemaphoreType.DMA,
            pltpu.SemaphoreType.DMA,
            pltpu.SemaphoreType.DMA,
            pltpu.SemaphoreType.DMA,
            pltpu.SemaphoreType.DMA,
            pltpu.SemaphoreType.DMA,
            pltpu.SemaphoreType.DMA,
            pltpu.SemaphoreType.DMA,
        ],
    )
    return fn(ids2, elt, ert)


def _gat_agg_body(ids_hbm, ee_hbm, feat_hbm, out_hbm, acc_sh,
                  ids_v0, ids_v1, ee_v0, ee_v1, idx_v0, idx_v1,
                  dst_v0, dst_v1, rows_v0, rows_v1, zeros_v,
                  semL0, semL1, semG0, semG1, semS0, semS1):
    c = lax.axis_index("c")
    s = lax.axis_index("s")
    base_row = s * ROWS_PER_SUB
    _zero_rows(zeros_v, D)

    ids_b = (ids_v0, ids_v1)
    ee_b = (ee_v0, ee_v1)
    idx_b = (idx_v0, idx_v1)
    dst_b = (dst_v0, dst_v1)
    rows_b = (rows_v0, rows_v1)
    semL = (semL0, semL1)
    semG = (semG0, semG1)
    semS = (semS0, semS1)

    for k in range(HPC):
        head = c * HPC + k
        _zero_slice(zeros_v, acc_sh, base_row)
        pltpu.async_copy(feat_hbm.at[pl.ds(0, CHUNK)], rows_v1, semS1)
        plsc.subcore_barrier()
        lane = jnp.broadcast_to(head, (L,))
        off = head * N

        def issue_loads(cid_expr, b):
            pltpu.async_copy(ids_hbm.at[cid_expr], ids_b[b], semL[b])
            pltpu.async_copy(ee_hbm.at[cid_expr], ee_b[b], semL[b])

        def wait_loads(cid_expr, b):
            pltpu.make_async_copy(ids_hbm.at[cid_expr], ids_b[b],
                                  semL[b]).wait()
            pltpu.make_async_copy(ee_hbm.at[cid_expr], ee_b[b],
                                  semL[b]).wait()

        def build_and_gather(b):
            for q in range(CHUNK // L):
                sl = pl.ds(q * L, L)
                idx_b[b][sl] = ids_b[b][sl] + off
                dst_b[b][sl] = ids_b[b][pl.ds(CHUNK + q * L, L)]
            pltpu.async_copy(feat_hbm.at[idx_b[b]], rows_b[b], semG[b])

        issue_loads(s, 0)
        wait_loads(s, 0)
        build_and_gather(0)
        issue_loads(s + NS, 1)

        def trip(t_expr, b):
            nb = 1 - b
            cid = s + t_expr * NS
            wait_loads(cid + NS, nb)
            pltpu.make_async_copy(rows_b[nb], acc_sh.at[dst_b[nb]],
                                  semS[nb]).wait()
            build_and_gather(nb)
            pltpu.make_async_copy(feat_hbm.at[idx_b[b]], rows_b[b],
                                  semG[b]).wait()

            @pl.loop(0, CHUNK, step=16)
            def _(jo):
                for u in range(2):
                    g = (jo >> 3) + u
                    for t in range(8):
                        wv = _take16(ee_b[b][g, pl.ds(t * L, L)], lane)
                        for tt in range(D // L):
                            sl = pl.ds(tt * L, L)
                            rows_b[b][jo + u * 8 + t, sl] = (
                                rows_b[b][jo + u * 8 + t, sl] * wv)

            pltpu.async_copy(rows_b[b], acc_sh.at[dst_b[b]], semS[b],
                             add=True)
            issue_loads(cid + 2 * NS, b)

        @pl.loop(0, TRIPS_S // 2)
        def _(i2):
            trip(i2 * 2, 0)
            trip(i2 * 2 + 1, 1)

        pltpu.make_async_copy(feat_hbm.at[idx_v0], rows_v0, semG0).wait()
        wait_loads(s + (TRIPS_S + 1) * NS, 1)
        pltpu.make_async_copy(rows_v1, acc_sh.at[dst_v1], semS1).wait()

        plsc.subcore_barrier()
        _copy_out(acc_sh, out_hbm, base_row, head * NPAD + base_row)
        plsc.subcore_barrier()


def _sc_gat_agg(ids2, ee, feat):
    fn = pl.kernel(
        _gat_agg_body,
        out_type=jax.ShapeDtypeStruct((H * NPAD, D), jnp.float32),
        mesh=_MESH,
        scratch_types=[
            pltpu.VMEM_SHARED((NPAD, D), jnp.float32),
            pltpu.VMEM((2 * CHUNK,), jnp.int32),
            pltpu.VMEM((2 * CHUNK,), jnp.int32),
            pltpu.VMEM((8, D), jnp.float32),
            pltpu.VMEM((8, D), jnp.float32),
            pltpu.VMEM((CHUNK,), jnp.int32),
            pltpu.VMEM((CHUNK,), jnp.int32),
            pltpu.VMEM((CHUNK,), jnp.int32),
            pltpu.VMEM((CHUNK,), jnp.int32),
            pltpu.VMEM((CHUNK, D), jnp.float32),
            pltpu.VMEM((CHUNK, D), jnp.float32),
            pltpu.VMEM((CHUNK, D), jnp.float32),
            pltpu.SemaphoreType.DMA,
            pltpu.SemaphoreType.DMA,
            pltpu.SemaphoreType.DMA,
            pltpu.SemaphoreType.DMA,
            pltpu.SemaphoreType.DMA,
            pltpu.SemaphoreType.DMA,
        ],
    )
    return fn(ids2, ee, feat)


def _gin_mlp_kernel(h_ref, a0_ref, a1_ref, d0_ref, d1_ref, eps_ref,
                    W1_ref, b1_ref, g_ref, beta_ref, W2_ref, b2_ref, o_ref):
    deg = jnp.maximum(d0_ref[:, 0:1] + d1_ref[:, 0:1], 1.0)
    agg = (a0_ref[...] + a1_ref[...]) / deg
    z = (1.0 + eps_ref[0, 0]) * h_ref[...] + agg
    z = jnp.maximum(jnp.dot(z, W1_ref[...],
                            preferred_element_type=jnp.float32) + b1_ref[...],
                    0.0)
    mu = jnp.mean(z, axis=0, keepdims=True)
    var = jnp.mean((z - mu) ** 2, axis=0, keepdims=True)
    z = (z - mu) / jnp.sqrt(var + 1e-5) * g_ref[...] + beta_ref[...]
    z = jnp.dot(z, W2_ref[...], preferred_element_type=jnp.float32) + b2_ref[...]
    o_ref[...] = jnp.maximum(z, 0.0)


def _tc_gin_mlp(h, a0, a1, d0, d1, eps, W1, b1, g, beta, W2, b2):
    return pl.pallas_call(
        _gin_mlp_kernel,
        out_shape=jax.ShapeDtypeStruct((N, HID), jnp.float32),
    )(h, a0, a1, d0, d1, eps.reshape(1, 1), W1, b1.reshape(1, HID),
      g.reshape(1, HID), beta.reshape(1, HID), W2, b2.reshape(1, HID))


def _feat_kernel(h_ref, W_ref, o_ref):
    o_ref[...] = jnp.dot(h_ref[...], W_ref[...],
                         preferred_element_type=jnp.float32)


def _tc_feat(h2, gat_W):
    return pl.pallas_call(
        _feat_kernel,
        grid=(H,),
        in_specs=[
            pl.BlockSpec((N, HID), lambda h: (0, 0)),
            pl.BlockSpec((HID, DH), lambda h: (0, h)),
        ],
        out_specs=pl.BlockSpec((N, DH), lambda h: (h, 0)),
        out_shape=jax.ShapeDtypeStruct((H * N, DH), jnp.float32),
    )(h2, gat_W)


def _tables_kernel(h2_ref, W_ref, al_ref, ar_ref, elt_ref, ert_ref):
    cols_l = []
    cols_r = []
    for h in range(H):
        Wh = W_ref[:, h * DH:(h + 1) * DH]
        cols_l.append(jnp.dot(Wh, al_ref[h, :][:, None],
                              preferred_element_type=jnp.float32))
        cols_r.append(jnp.dot(Wh, ar_ref[h, :][:, None],
                              preferred_element_type=jnp.float32))
    Wal = jnp.concatenate(cols_l, axis=1)
    War = jnp.concatenate(cols_r, axis=1)
    el = jnp.dot(h2_ref[...], Wal, preferred_element_type=jnp.float32)
    er = jnp.dot(h2_ref[...], War, preferred_element_type=jnp.float32)
    elmax = jnp.max(el, axis=0, keepdims=True)
    t = elmax + er
    shift = jnp.where(t >= 0, t, 0.2 * t)
    zero = jnp.zeros((NPAD, D - L), jnp.float32)
    zero16 = jnp.zeros((NPAD - N, L), jnp.float32)
    elt_ref[:N, :H] = el
    elt_ref[:N, H:L] = el
    elt_ref[N:, :L] = zero16
    elt_ref[:, L:] = zero
    ert_ref[:N, :H] = er
    ert_ref[:N, H:L] = shift
    ert_ref[N:, :L] = zero16
    ert_ref[:, L:] = zero


def _tc_tables(h2, gat_W, gat_al, gat_ar):
    return pl.pallas_call(
        _tables_kernel,
        out_shape=(jax.ShapeDtypeStruct((NPAD, D), jnp.float32),
                   jax.ShapeDtypeStruct((NPAD, D), jnp.float32)),
    )(h2, gat_W, gat_al, gat_ar)


def _final_kernel(h1_ref, h2_ref, agg_ref, z0_ref, z1_ref, b_ref, o_ref):
    h = pl.program_id(0)

    @pl.when(h == 0)
    def _():
        o_ref[:, :D] = h1_ref[...]
        o_ref[:, D:2 * D] = h2_ref[...]
        o_ref[:, 2 * D:] = jnp.zeros((N, D), jnp.float32)

    zc = (z0_ref[0, 0, :N] + z1_ref[0, 0, :N]).reshape(N, 1)
    denom = jnp.where(zc > 0, zc, 1.0)
    bias = b_ref[0, 0, :].reshape(1, DH)
    contrib = jnp.maximum(agg_ref[:N, :] / denom + bias, 0.0)
    o_ref[:, 2 * D:] += contrib * (1.0 / H)


def _tc_final(h1, h2, aggout, z0t, z1t, gat_b):
    return pl.pallas_call(
        _final_kernel,
        grid=(H,),
        in_specs=[
            pl.BlockSpec((N, D), lambda h: (0, 0)),
            pl.BlockSpec((N, D), lambda h: (0, 0)),
            pl.BlockSpec((NPAD, D), lambda h: (h, 0)),
            pl.BlockSpec((1, 1, NPAD), lambda h: (h, 0, 0)),
            pl.BlockSpec((1, 1, NPAD), lambda h: (h, 0, 0)),
            pl.BlockSpec((1, 1, DH), lambda h: (h, 0, 0)),
        ],
        out_specs=pl.BlockSpec((N, 3 * D), lambda h: (0, 0)),
        out_shape=jax.ShapeDtypeStruct((N, 3 * D), jnp.float32),
    )(h1, h2, aggout, z0t, z1t, gat_b)


def kernel(x, edge_index, gin1_W1, gin1_b1, gin1_g, gin1_beta, gin1_W2,
           gin1_b2, gin1_eps, gin2_W1, gin2_b1, gin2_g, gin2_beta, gin2_W2,
           gin2_b2, gin2_eps, gat_W, gat_al, gat_ar, gat_b):
    src = edge_index[0]
    dst = edge_index[1]
    npadE = EPAD - E
    srcp = jnp.concatenate([src, jnp.zeros((npadE,), jnp.int32)])
    dstp = jnp.concatenate([dst, jnp.full((npadE,), N, jnp.int32)])
    idsrow = jnp.concatenate([srcp.reshape(TCHP, CHUNK),
                              dstp.reshape(TCHP, CHUNK)], axis=1)
    ids2 = jnp.concatenate(
        [idsrow, jnp.zeros((TCHX - TCHP, 2 * CHUNK), jnp.int32)])


    degp = _sc_deg(ids2)
    d0 = degp[:N, :L]
    d1 = degp[NPAD:NPAD + N, :L]

    ssum1 = _sc_gin_agg(ids2, x)
    h1 = _tc_gin_mlp(x, ssum1[:N, :], ssum1[NPAD:NPAD + N, :], d0, d1,
                     gin1_eps, gin1_W1, gin1_b1, gin1_g, gin1_beta,
                     gin1_W2, gin1_b2)

    ssum2 = _sc_gin_agg(ids2, h1)
    h2 = _tc_gin_mlp(h1, ssum2[:N, :], ssum2[NPAD:NPAD + N, :], d0, d1,
                     gin2_eps, gin2_W1, gin2_b1, gin2_g, gin2_beta,
                     gin2_W2, gin2_b2)

    feat = _tc_feat(h2, gat_W)
    elt, ert = _tc_tables(h2, gat_W, gat_al, gat_ar)
    ee, zpart = _sc_gat_edge(ids2, elt, ert)
    aggout = _sc_gat_agg(ids2, ee, feat)

    z0t = zpart[:NPAD, :H].T.reshape(H, 1, NPAD)
    z1t = zpart[NPAD:, :H].T.reshape(H, 1, NPAD)
    bt = gat_b.reshape(H, 1, DH)
    return _tc_final(h1, h2, aggout, z0t, z1t, bt)
